# Initial kernel scaffold; baseline (speedup 1.0000x reference)
import jax
import jax.numpy as jnp
from jax.experimental import pallas as pl


def kernel(input_memories, input_memories_origin_idx, initial_states, target_token_ids, copyable_elements_idxs, copyable_elements_sample_idxs, target_lengths, emb, w_ih, w_hh, b_ih, b_hh, w_std_attn, w_copy_attn, hidden_to_vocab, vocab_bias):
    raise NotImplementedError("write your pallas kernel here")


# speedup vs baseline: 4.0626x; 4.0626x over previous
import functools
import jax
import jax.numpy as jnp
from jax.experimental import pallas as pl
from jax.experimental.pallas import tpu as pltpu

_V = 10000
_E = 128
_H = 128
_D = 256
_B = 128
_S = 21
_L = 20
_NI = 8192
_NC = 4096
_UNK = 3
_NEG = float("-inf")

_pc = pl.pallas_call

_TI = 512
_VP = 10240
_VT = 1024
_CT = 1024
_BL = _B * _L


def _reps_body(x_ref, w_ref, o_ref):
    o_ref[...] = jnp.dot(x_ref[...], w_ref[...],
                         preferred_element_type=jnp.float32)


def _gru_body(x_ref, h0_ref, wih_ref, whh_ref, bih_ref, bhh_ref,
              os_ref, h_scr):
    l = pl.program_id(0)

    @pl.when(l == 0)
    def _():
        h_scr[...] = h0_ref[...]

    h = h_scr[...]
    x = x_ref[0]
    gi = jnp.dot(x, wih_ref[...], preferred_element_type=jnp.float32) \
        + bih_ref[0:1, :]
    gh = jnp.dot(h, whh_ref[...], preferred_element_type=jnp.float32) \
        + bhh_ref[0:1, :]
    r = jax.nn.sigmoid(gi[:, :_H] + gh[:, :_H])
    z = jax.nn.sigmoid(gi[:, _H:2 * _H] + gh[:, _H:2 * _H])
    n = jnp.tanh(gi[:, 2 * _H:] + r * gh[:, 2 * _H:])
    h_new = (1.0 - z) * n + z * h
    h_scr[...] = h_new
    os_ref[0] = h_new


def _scores_body(reps_ref, org_ref, osf_ref, ostd_ref, ocpy_ref):
    iota_b = jax.lax.broadcasted_iota(jnp.int32, (_TI, _B), 1)
    onehot = (org_ref[...] == iota_b).astype(jnp.float32)
    g = jnp.dot(onehot, osf_ref[...], preferred_element_type=jnp.float32)
    g3 = g.reshape(_TI, _L, _H)
    std = reps_ref[:, :_H]
    cpy = reps_ref[:, _H:]
    ostd_ref[...] = jnp.sum(g3 * std[:, None, :], axis=2)
    ocpy_ref[...] = jnp.sum(g3 * cpy[:, None, :], axis=2)


def _segstats_body(sstd_ref, scpy_ref, org_ref, stdrep_ref,
                   so_ref, tc_ref, mstd_scr, mcpy_scr):
    iota_b = jax.lax.broadcasted_iota(jnp.int32, (_NI, _B), 1)
    onehot_b = org_ref[...] == iota_b
    onehot = onehot_b.astype(jnp.float32)
    neg = jnp.float32(_NEG)
    for l in range(_L):
        col_s = sstd_ref[:, l:l + 1]
        col_c = scpy_ref[:, l:l + 1]
        mstd_scr[l:l + 1, :] = jnp.max(
            jnp.where(onehot_b, col_s, neg), axis=0, keepdims=True)
        mcpy_scr[l:l + 1, :] = jnp.max(
            jnp.where(onehot_b, col_c, neg), axis=0, keepdims=True)
    mstd = mstd_scr[...]
    mcpy = mcpy_scr[...]
    mstd_safe = jnp.where(jnp.isfinite(mstd), mstd, 0.0)
    mcpy_safe = jnp.where(jnp.isfinite(mcpy), mcpy, 0.0)
    dn = (((1,), (1,)), ((), ()))
    d0 = (((0,), (0,)), ((), ()))
    g_ms = jax.lax.dot_general(onehot, mstd_safe, dn,
                               preferred_element_type=jnp.float32)
    g_mc = jax.lax.dot_general(onehot, mcpy_safe, dn,
                               preferred_element_type=jnp.float32)
    exp_s = jnp.exp(sstd_ref[...] - g_ms)
    exp_c = jnp.exp(scpy_ref[...] - g_mc)
    ssum_lb = jax.lax.dot_general(exp_s, onehot, d0,
                                  preferred_element_type=jnp.float32)
    csum_lb = jax.lax.dot_general(exp_c, onehot, d0,
                                  preferred_element_type=jnp.float32)
    logden_lb = jnp.log(jnp.where(ssum_lb > 0, ssum_lb, 1.0))
    g_ld = jax.lax.dot_general(onehot, logden_lb, dn,
                               preferred_element_type=jnp.float32)
    p = exp_s * jnp.exp(-g_ld)
    stdrep = stdrep_ref[...]
    for l in range(_L):
        tmp = p[:, l:l + 1] * stdrep
        so_ref[l] = jax.lax.dot_general(onehot, tmp, d0,
                                        preferred_element_type=jnp.float32)
    tc_ref[...] = jnp.where(
        jnp.isfinite(mcpy),
        mcpy_safe + jnp.log(jnp.where(csum_lb > 0, csum_lb, 1.0)),
        neg)


def _vocab_body(cat_ref, h2v_ref, emb_ref, bias_ref, gold_ref, tc_ref,
                norm_ref, gold_out_ref, proj_scr, m_scr, s_scr, g_scr):
    v = pl.program_id(0)
    neg = jnp.float32(_NEG)

    @pl.when(v == 0)
    def _():
        proj_scr[...] = jnp.dot(cat_ref[...], h2v_ref[...],
                                preferred_element_type=jnp.float32)
        m_scr[...] = jnp.full((_BL, 128), neg, jnp.float32)
        s_scr[...] = jnp.zeros((_BL, 128), jnp.float32)
        g_scr[...] = jnp.zeros((_BL, 128), jnp.float32)

    dnT = (((1,), (1,)), ((), ()))
    scores = jax.lax.dot_general(proj_scr[...], emb_ref[...], dnT,
                                 preferred_element_type=jnp.float32)
    scores = scores + bias_ref[0:1, :]
    col = jax.lax.broadcasted_iota(jnp.int32, (_BL, _VT), 1) + v * _VT
    match = col == gold_ref[:, 0:1]
    g_scr[...] += jnp.sum(jnp.where(match, scores, 0.0), axis=1,
                          keepdims=True)
    m_old = m_scr[...]
    t_max = jnp.max(scores, axis=1, keepdims=True)
    m_new = jnp.maximum(m_old, t_max)
    p = jnp.sum(jnp.exp(scores - m_new[:, 0:1]), axis=1, keepdims=True)
    s_scr[...] = s_scr[...] * jnp.exp(m_old - m_new) + p
    m_scr[...] = m_new

    @pl.when(v == pl.num_programs(0) - 1)
    def _():
        m = m_scr[...]
        s = s_scr[...]
        tc = tc_ref[...]
        m_f = jnp.maximum(m, tc)
        s_f = s * jnp.exp(m - m_f) + jnp.exp(tc - m_f)
        norm_ref[...] = m_f + jnp.log(s_f)
        gold_out_ref[...] = g_scr[...]


def _clp_body(scpy_ref, org_ref, norm_ref, clp_ref):
    iota_b = jax.lax.broadcasted_iota(jnp.int32, (_NI, _B), 1)
    onehot = (org_ref[...] == iota_b).astype(jnp.float32)
    g_norm = jnp.dot(onehot, norm_ref[...],
                     preferred_element_type=jnp.float32)
    clp_ref[...] = scpy_ref[...] - g_norm


def _copyseg_body(sel_ref, sidx_ref, cc_ref, nv_ref,
                  mx_scr, sv_scr, nv_scr):
    p = pl.program_id(0)
    t = pl.program_id(1)
    neg = jnp.float32(_NEG)

    @pl.when((p == 0) & (t == 0))
    def _():
        mx_scr[...] = jnp.full((8, _BL), neg, jnp.float32)
        sv_scr[...] = jnp.zeros((8, _BL), jnp.float32)
        nv_scr[...] = jnp.zeros((8, _BL), jnp.float32)

    iota_s = jax.lax.broadcasted_iota(jnp.int32, (_CT, _BL), 1)
    onehot_b = sidx_ref[:, 0:1] == iota_s
    sel_col = sel_ref[:, 0:1]

    @pl.when(p == 0)
    def _():
        masked = jnp.where(onehot_b, sel_col, neg)
        mx_scr[0:1, :] = jnp.maximum(
            mx_scr[0:1, :], jnp.max(masked, axis=0, keepdims=True))
        nv_scr[0:1, :] += jnp.sum(onehot_b.astype(jnp.float32), axis=0,
                                  keepdims=True)

    @pl.when(p == 1)
    def _():
        gmx = jnp.max(jnp.where(onehot_b, mx_scr[0:1, :], neg), axis=1,
                      keepdims=True)
        e = jnp.exp(sel_col - gmx)
        sv_scr[0:1, :] += jnp.sum(jnp.where(onehot_b, e, 0.0), axis=0,
                                  keepdims=True)

    @pl.when((p == 1) & (t == pl.num_programs(1) - 1))
    def _():
        mx = mx_scr[0:1, :]
        sv = sv_scr[0:1, :]
        cc = jnp.where(jnp.isfinite(mx),
                       mx + jnp.log(jnp.where(sv > 0, sv, 1.0)),
                       neg)
        cc_ref[...] = jnp.broadcast_to(cc, (8, _BL))
        nv_ref[...] = jnp.broadcast_to(nv_scr[0:1, :], (8, _BL))


def _loss_body(glp_ref, cc_ref, nv_ref, tt_ref, len_ref, out_ref):
    neg = jnp.float32(_NEG)
    glp = glp_ref[...]
    cc = cc_ref[...]
    mask = (nv_ref[...] > 0) & (tt_ref[...] == _UNK)
    cg = jnp.where(mask, neg, glp)
    m2 = jnp.maximum(cg, cc)
    fin = jnp.isfinite(m2)
    m2s = jnp.where(fin, m2, 0.0)
    any_c = jnp.where(
        fin, m2s + jnp.log(jnp.exp(cg - m2s) + jnp.exp(cc - m2s)), neg)
    iota_l = jax.lax.broadcasted_iota(jnp.int32, (_B, _L), 1)
    lm = (iota_l < len_ref[...]).astype(jnp.float32)
    per_seq = jnp.sum(any_c * lm, axis=1, keepdims=True) / \
        jnp.sum(lm, axis=1, keepdims=True)
    loss = -jnp.sum(per_seq) / _B
    out_ref[...] = jnp.broadcast_to(loss, (8, 128))


def kernel(input_memories, input_memories_origin_idx, initial_states,
           target_token_ids, copyable_elements_idxs,
           copyable_elements_sample_idxs, target_lengths, emb, w_ih, w_hh,
           b_ih, b_hh, w_std_attn, w_copy_attn, hidden_to_vocab,
           vocab_bias):
    f32 = jnp.float32

    w_cat_t = jnp.concatenate([w_std_attn, w_copy_attn], axis=0).T
    reps = _pc(
        _reps_body,
        grid=(_NI // _TI,),
        in_specs=[
            pl.BlockSpec((_TI, _D), lambda t: (t, 0)),
            pl.BlockSpec((_D, 2 * _H), lambda t: (0, 0)),
        ],
        out_specs=pl.BlockSpec((_TI, 2 * _H), lambda t: (t, 0)),
        out_shape=jax.ShapeDtypeStruct((_NI, 2 * _H), f32),
    )(input_memories, w_cat_t)

    input_tok = target_token_ids[:, :-1]
    x_lbe = jnp.take(emb, input_tok, axis=0).transpose(1, 0, 2)
    bih8 = jnp.broadcast_to(b_ih[None, :], (8, 3 * _H))
    bhh8 = jnp.broadcast_to(b_hh[None, :], (8, 3 * _H))
    os_lbh = _pc(
        _gru_body,
        grid=(_L,),
        in_specs=[
            pl.BlockSpec((1, _B, _E), lambda l: (l, 0, 0)),
            pl.BlockSpec((_B, _H), lambda l: (0, 0)),
            pl.BlockSpec((_E, 3 * _H), lambda l: (0, 0)),
            pl.BlockSpec((_H, 3 * _H), lambda l: (0, 0)),
            pl.BlockSpec((8, 3 * _H), lambda l: (0, 0)),
            pl.BlockSpec((8, 3 * _H), lambda l: (0, 0)),
        ],
        out_specs=pl.BlockSpec((1, _B, _H), lambda l: (l, 0, 0)),
        out_shape=jax.ShapeDtypeStruct((_L, _B, _H), f32),
        scratch_shapes=[pltpu.VMEM((_B, _H), f32)],
    )(x_lbe, initial_states, w_ih.T, w_hh.T, bih8, bhh8)

    org_lanes = jnp.broadcast_to(
        input_memories_origin_idx[:, None], (_NI, _B))
    osf = os_lbh.transpose(1, 0, 2).reshape(_B, _L * _H)
    s_std, s_cpy = _pc(
        _scores_body,
        grid=(_NI // _TI,),
        in_specs=[
            pl.BlockSpec((_TI, 2 * _H), lambda t: (t, 0)),
            pl.BlockSpec((_TI, _B), lambda t: (t, 0)),
            pl.BlockSpec((_B, _L * _H), lambda t: (0, 0)),
        ],
        out_specs=[
            pl.BlockSpec((_TI, _L), lambda t: (t, 0)),
            pl.BlockSpec((_TI, _L), lambda t: (t, 0)),
        ],
        out_shape=[
            jax.ShapeDtypeStruct((_NI, _L), f32),
            jax.ShapeDtypeStruct((_NI, _L), f32),
        ],
    )(reps, org_lanes, osf)

    so_lbh, tc_lb = _pc(
        _segstats_body,
        in_specs=[
            pl.BlockSpec((_NI, _L), lambda: (0, 0)),
            pl.BlockSpec((_NI, _L), lambda: (0, 0)),
            pl.BlockSpec((_NI, _B), lambda: (0, 0)),
            pl.BlockSpec((_NI, _H), lambda: (0, 0)),
        ],
        out_specs=[
            pl.BlockSpec((_L, _B, _H), lambda: (0, 0, 0)),
            pl.BlockSpec((_L, _B), lambda: (0, 0)),
        ],
        out_shape=[
            jax.ShapeDtypeStruct((_L, _B, _H), f32),
            jax.ShapeDtypeStruct((_L, _B), f32),
        ],
        scratch_shapes=[
            pltpu.VMEM((_L, _B), f32),
            pltpu.VMEM((_L, _B), f32),
        ],
    )(s_std, s_cpy, org_lanes, reps[:, :_H])

    cat = jnp.concatenate(
        [so_lbh.transpose(1, 0, 2), os_lbh.transpose(1, 0, 2)],
        axis=-1).reshape(_BL, 2 * _H)
    gold = target_token_ids[:, 1:].reshape(_BL)
    gold_lanes = jnp.broadcast_to(gold[:, None], (_BL, 128))
    tc_bl = tc_lb.T.reshape(_BL)
    tc_lanes = jnp.broadcast_to(tc_bl[:, None], (_BL, 128))
    emb_p = jnp.concatenate(
        [emb, jnp.zeros((_VP - _V, _E), f32)], axis=0)
    bias_p = jnp.concatenate(
        [vocab_bias, jnp.full((_VP - _V,), -1e30, f32)])
    bias8 = jnp.broadcast_to(bias_p[None, :], (8, _VP))
    norm_w, gold_w = _pc(
        _vocab_body,
        grid=(_VP // _VT,),
        in_specs=[
            pl.BlockSpec((_BL, 2 * _H), lambda v: (0, 0)),
            pl.BlockSpec((2 * _H, _E), lambda v: (0, 0)),
            pl.BlockSpec((_VT, _E), lambda v: (v, 0)),
            pl.BlockSpec((8, _VT), lambda v: (0, v)),
            pl.BlockSpec((_BL, 128), lambda v: (0, 0)),
            pl.BlockSpec((_BL, 128), lambda v: (0, 0)),
        ],
        out_specs=[
            pl.BlockSpec((_BL, 128), lambda v: (0, 0)),
            pl.BlockSpec((_BL, 128), lambda v: (0, 0)),
        ],
        out_shape=[
            jax.ShapeDtypeStruct((_BL, 128), f32),
            jax.ShapeDtypeStruct((_BL, 128), f32),
        ],
        scratch_shapes=[
            pltpu.VMEM((_BL, _H), f32),
            pltpu.VMEM((_BL, 128), f32),
            pltpu.VMEM((_BL, 128), f32),
            pltpu.VMEM((_BL, 128), f32),
        ],
    )(cat, hidden_to_vocab, emb_p, bias8, gold_lanes, tc_lanes)
    norm_bl = norm_w[:, 0].reshape(_B, _L)

    clp = _pc(
        _clp_body,
        in_specs=[
            pl.BlockSpec((_NI, _L), lambda: (0, 0)),
            pl.BlockSpec((_NI, _B), lambda: (0, 0)),
            pl.BlockSpec((_B, _L), lambda: (0, 0)),
        ],
        out_specs=pl.BlockSpec((_NI, _L), lambda: (0, 0)),
        out_shape=jax.ShapeDtypeStruct((_NI, _L), f32),
    )(s_cpy, org_lanes, norm_bl)

    sel = clp.reshape(-1)[copyable_elements_idxs]
    sel_lanes = jnp.broadcast_to(sel[:, None], (_NC, 128))
    sidx_lanes = jnp.broadcast_to(
        copyable_elements_sample_idxs[:, None], (_NC, 128))

    cc_w, nv_w = _pc(
        _copyseg_body,
        grid=(2, _NC // _CT),
        in_specs=[
            pl.BlockSpec((_CT, 128), lambda p, t: (t, 0)),
            pl.BlockSpec((_CT, 128), lambda p, t: (t, 0)),
        ],
        out_specs=[
            pl.BlockSpec((8, _BL), lambda p, t: (0, 0)),
            pl.BlockSpec((8, _BL), lambda p, t: (0, 0)),
        ],
        out_shape=[
            jax.ShapeDtypeStruct((8, _BL), f32),
            jax.ShapeDtypeStruct((8, _BL), f32),
        ],
        scratch_shapes=[
            pltpu.VMEM((8, _BL), f32),
            pltpu.VMEM((8, _BL), f32),
            pltpu.VMEM((8, _BL), f32),
        ],
    )(sel_lanes, sidx_lanes)

    glp_bl = (gold_w[:, 0] - norm_w[:, 0]).reshape(_B, _L)
    cc_bl = cc_w[0].reshape(_B, _L)
    nv_bl = nv_w[0].reshape(_B, _L)
    tt = target_token_ids[:, 1:]
    len_b = jnp.broadcast_to(target_lengths[:, None], (_B, _L))
    out = _pc(
        _loss_body,
        in_specs=[
            pl.BlockSpec((_B, _L), lambda: (0, 0)),
            pl.BlockSpec((_B, _L), lambda: (0, 0)),
            pl.BlockSpec((_B, _L), lambda: (0, 0)),
            pl.BlockSpec((_B, _L), lambda: (0, 0)),
            pl.BlockSpec((_B, _L), lambda: (0, 0)),
        ],
        out_specs=pl.BlockSpec((8, 128), lambda: (0, 0)),
        out_shape=jax.ShapeDtypeStruct((8, 128), f32),
    )(glp_bl, cc_bl, nv_bl, tt, len_b)
    return out[0, 0]


# speedup vs baseline: 4.1129x; 1.0124x over previous
import functools
import jax
import jax.numpy as jnp
from jax import lax
from jax.experimental import pallas as pl
from jax.experimental.pallas import tpu as pltpu
from jax.experimental.pallas import tpu_sc as plsc

_V = 10000
_E = 128
_H = 128
_D = 256
_B = 128
_S = 21
_L = 20
_NI = 8192
_NC = 4096
_UNK = 3
_NEG = float("-inf")

_pc = pl.pallas_call

_TI = 512
_VP = 10240
_VT = 1024
_CT = 1024
_BL = _B * _L


def _reps_body(x_ref, w_ref, o_ref):
    o_ref[...] = jnp.dot(x_ref[...], w_ref[...],
                         preferred_element_type=jnp.float32)


def _gru_body(x_ref, h0_ref, wih_ref, whh_ref, bih_ref, bhh_ref,
              os_ref, h_scr):
    l = pl.program_id(0)

    @pl.when(l == 0)
    def _():
        h_scr[...] = h0_ref[...]

    h = h_scr[...]
    x = x_ref[0]
    gi = jnp.dot(x, wih_ref[...], preferred_element_type=jnp.float32) \
        + bih_ref[0:1, :]
    gh = jnp.dot(h, whh_ref[...], preferred_element_type=jnp.float32) \
        + bhh_ref[0:1, :]
    r = jax.nn.sigmoid(gi[:, :_H] + gh[:, :_H])
    z = jax.nn.sigmoid(gi[:, _H:2 * _H] + gh[:, _H:2 * _H])
    n = jnp.tanh(gi[:, 2 * _H:] + r * gh[:, 2 * _H:])
    h_new = (1.0 - z) * n + z * h
    h_scr[...] = h_new
    os_ref[0] = h_new


def _scores_body(reps_ref, org_ref, osf_ref, ostd_ref, ocpy_ref):
    iota_b = jax.lax.broadcasted_iota(jnp.int32, (_TI, _B), 1)
    onehot = (org_ref[...] == iota_b).astype(jnp.float32)
    g = jnp.dot(onehot, osf_ref[...], preferred_element_type=jnp.float32)
    g3 = g.reshape(_TI, _L, _H)
    std = reps_ref[:, :_H]
    cpy = reps_ref[:, _H:]
    ostd_ref[...] = jnp.sum(g3 * std[:, None, :], axis=2)
    ocpy_ref[...] = jnp.sum(g3 * cpy[:, None, :], axis=2)


def _segstats_body(sstd_ref, scpy_ref, org_ref, stdrep_ref,
                   so_ref, tc_ref, mstd_scr, mcpy_scr):
    iota_b = jax.lax.broadcasted_iota(jnp.int32, (_NI, _B), 1)
    onehot_b = org_ref[...] == iota_b
    onehot = onehot_b.astype(jnp.float32)
    neg = jnp.float32(_NEG)
    for l in range(_L):
        col_s = sstd_ref[:, l:l + 1]
        col_c = scpy_ref[:, l:l + 1]
        mstd_scr[l:l + 1, :] = jnp.max(
            jnp.where(onehot_b, col_s, neg), axis=0, keepdims=True)
        mcpy_scr[l:l + 1, :] = jnp.max(
            jnp.where(onehot_b, col_c, neg), axis=0, keepdims=True)
    mstd = mstd_scr[...]
    mcpy = mcpy_scr[...]
    mstd_safe = jnp.where(jnp.isfinite(mstd), mstd, 0.0)
    mcpy_safe = jnp.where(jnp.isfinite(mcpy), mcpy, 0.0)
    dn = (((1,), (1,)), ((), ()))
    d0 = (((0,), (0,)), ((), ()))
    g_ms = jax.lax.dot_general(onehot, mstd_safe, dn,
                               preferred_element_type=jnp.float32)
    g_mc = jax.lax.dot_general(onehot, mcpy_safe, dn,
                               preferred_element_type=jnp.float32)
    exp_s = jnp.exp(sstd_ref[...] - g_ms)
    exp_c = jnp.exp(scpy_ref[...] - g_mc)
    ssum_lb = jax.lax.dot_general(exp_s, onehot, d0,
                                  preferred_element_type=jnp.float32)
    csum_lb = jax.lax.dot_general(exp_c, onehot, d0,
                                  preferred_element_type=jnp.float32)
    logden_lb = jnp.log(jnp.where(ssum_lb > 0, ssum_lb, 1.0))
    g_ld = jax.lax.dot_general(onehot, logden_lb, dn,
                               preferred_element_type=jnp.float32)
    p = exp_s * jnp.exp(-g_ld)
    stdrep = stdrep_ref[...]
    for l in range(_L):
        tmp = p[:, l:l + 1] * stdrep
        so_ref[l] = jax.lax.dot_general(onehot, tmp, d0,
                                        preferred_element_type=jnp.float32)
    tc_ref[...] = jnp.where(
        jnp.isfinite(mcpy),
        mcpy_safe + jnp.log(jnp.where(csum_lb > 0, csum_lb, 1.0)),
        neg)


def _vocab_body(cat_ref, h2v_ref, emb_ref, bias_ref, gold_ref, tc_ref,
                norm_ref, gold_out_ref, proj_scr, m_scr, s_scr, g_scr):
    v = pl.program_id(0)
    neg = jnp.float32(_NEG)

    @pl.when(v == 0)
    def _():
        proj_scr[...] = jnp.dot(cat_ref[...], h2v_ref[...],
                                preferred_element_type=jnp.float32)
        m_scr[...] = jnp.full((_BL, 128), neg, jnp.float32)
        s_scr[...] = jnp.zeros((_BL, 128), jnp.float32)
        g_scr[...] = jnp.zeros((_BL, 128), jnp.float32)

    dnT = (((1,), (1,)), ((), ()))
    scores = jax.lax.dot_general(proj_scr[...], emb_ref[...], dnT,
                                 preferred_element_type=jnp.float32)
    scores = scores + bias_ref[0:1, :]
    col = jax.lax.broadcasted_iota(jnp.int32, (_BL, _VT), 1) + v * _VT
    match = col == gold_ref[:, 0:1]
    g_scr[...] += jnp.sum(jnp.where(match, scores, 0.0), axis=1,
                          keepdims=True)
    m_old = m_scr[...]
    t_max = jnp.max(scores, axis=1, keepdims=True)
    m_new = jnp.maximum(m_old, t_max)
    p = jnp.sum(jnp.exp(scores - m_new[:, 0:1]), axis=1, keepdims=True)
    s_scr[...] = s_scr[...] * jnp.exp(m_old - m_new) + p
    m_scr[...] = m_new

    @pl.when(v == pl.num_programs(0) - 1)
    def _():
        m = m_scr[...]
        s = s_scr[...]
        tc = tc_ref[...]
        m_f = jnp.maximum(m, tc)
        s_f = s * jnp.exp(m - m_f) + jnp.exp(tc - m_f)
        norm_ref[...] = m_f + jnp.log(s_f)
        gold_out_ref[...] = g_scr[...]


def _clp_body(scpy_ref, org_ref, norm_ref, clp_ref):
    iota_b = jax.lax.broadcasted_iota(jnp.int32, (_NI, _B), 1)
    onehot = (org_ref[...] == iota_b).astype(jnp.float32)
    g_norm = jnp.dot(onehot, norm_ref[...],
                     preferred_element_type=jnp.float32)
    clp = scpy_ref[...] - g_norm
    clp_ref[...] = jnp.concatenate(
        [clp, jnp.zeros((_NI, 128 - _L), jnp.float32)], axis=1)


def _sc_gather(clp_pad, rows):
    info = plsc.get_sparse_core_info()
    ncores, nsub = info.num_cores, info.num_subcores
    nw = ncores * nsub
    per_w = _NC // nw
    mesh = plsc.VectorSubcoreMesh(core_axis_name="c", subcore_axis_name="s")

    @functools.partial(
        pl.kernel, mesh=mesh,
        out_type=jax.ShapeDtypeStruct((_NC, 128), jnp.float32),
        scratch_types=[
            pltpu.VMEM((per_w,), jnp.int32),
            pltpu.VMEM((per_w, 128), jnp.float32),
            pltpu.SemaphoreType.DMA,
        ],
    )
    def k(clp_hbm, row_hbm, out_hbm, row_v, rows_v, sem):
        wid = lax.axis_index("s") * ncores + lax.axis_index("c")
        base = wid * per_w
        pltpu.sync_copy(row_hbm.at[pl.ds(base, per_w)], row_v)
        pltpu.async_copy(clp_hbm.at[row_v], rows_v, sem).wait()
        pltpu.sync_copy(rows_v, out_hbm.at[pl.ds(base, per_w)])

    return k(clp_pad, rows)


def _copyseg_body(selr_ref, colh_ref, sidx_ref, cc_ref, nv_ref,
                  mx_scr, sv_scr, nv_scr):
    p = pl.program_id(0)
    t = pl.program_id(1)
    neg = jnp.float32(_NEG)
    iota32 = jax.lax.broadcasted_iota(jnp.int32, (_CT, 128), 1)
    sel_col = jnp.sum(
        jnp.where(iota32 == colh_ref[...], selr_ref[...], 0.0),
        axis=1, keepdims=True)

    @pl.when((p == 0) & (t == 0))
    def _():
        mx_scr[...] = jnp.full((8, _BL), neg, jnp.float32)
        sv_scr[...] = jnp.zeros((8, _BL), jnp.float32)
        nv_scr[...] = jnp.zeros((8, _BL), jnp.float32)

    iota_s = jax.lax.broadcasted_iota(jnp.int32, (_CT, _BL), 1)
    onehot_b = sidx_ref[:, 0:1] == iota_s

    @pl.when(p == 0)
    def _():
        masked = jnp.where(onehot_b, sel_col, neg)
        mx_scr[0:1, :] = jnp.maximum(
            mx_scr[0:1, :], jnp.max(masked, axis=0, keepdims=True))
        nv_scr[0:1, :] += jnp.sum(onehot_b.astype(jnp.float32), axis=0,
                                  keepdims=True)

    @pl.when(p == 1)
    def _():
        gmx = jnp.max(jnp.where(onehot_b, mx_scr[0:1, :], neg), axis=1,
                      keepdims=True)
        e = jnp.exp(sel_col - gmx)
        sv_scr[0:1, :] += jnp.sum(jnp.where(onehot_b, e, 0.0), axis=0,
                                  keepdims=True)

    @pl.when((p == 1) & (t == pl.num_programs(1) - 1))
    def _():
        mx = mx_scr[0:1, :]
        sv = sv_scr[0:1, :]
        cc = jnp.where(jnp.isfinite(mx),
                       mx + jnp.log(jnp.where(sv > 0, sv, 1.0)),
                       neg)
        cc_ref[...] = jnp.broadcast_to(cc, (8, _BL))
        nv_ref[...] = jnp.broadcast_to(nv_scr[0:1, :], (8, _BL))


def _loss_body(glp_ref, cc_ref, nv_ref, tt_ref, len_ref, out_ref):
    neg = jnp.float32(_NEG)
    glp = glp_ref[...]
    cc = cc_ref[...]
    mask = (nv_ref[...] > 0) & (tt_ref[...] == _UNK)
    cg = jnp.where(mask, neg, glp)
    m2 = jnp.maximum(cg, cc)
    fin = jnp.isfinite(m2)
    m2s = jnp.where(fin, m2, 0.0)
    any_c = jnp.where(
        fin, m2s + jnp.log(jnp.exp(cg - m2s) + jnp.exp(cc - m2s)), neg)
    iota_l = jax.lax.broadcasted_iota(jnp.int32, (_B, _L), 1)
    lm = (iota_l < len_ref[...]).astype(jnp.float32)
    per_seq = jnp.sum(any_c * lm, axis=1, keepdims=True) / \
        jnp.sum(lm, axis=1, keepdims=True)
    loss = -jnp.sum(per_seq) / _B
    out_ref[...] = jnp.broadcast_to(loss, (8, 128))


def kernel(input_memories, input_memories_origin_idx, initial_states,
           target_token_ids, copyable_elements_idxs,
           copyable_elements_sample_idxs, target_lengths, emb, w_ih, w_hh,
           b_ih, b_hh, w_std_attn, w_copy_attn, hidden_to_vocab,
           vocab_bias):
    f32 = jnp.float32

    w_cat_t = jnp.concatenate([w_std_attn, w_copy_attn], axis=0).T
    reps = _pc(
        _reps_body,
        grid=(_NI // _TI,),
        in_specs=[
            pl.BlockSpec((_TI, _D), lambda t: (t, 0)),
            pl.BlockSpec((_D, 2 * _H), lambda t: (0, 0)),
        ],
        out_specs=pl.BlockSpec((_TI, 2 * _H), lambda t: (t, 0)),
        out_shape=jax.ShapeDtypeStruct((_NI, 2 * _H), f32),
    )(input_memories, w_cat_t)

    input_tok = target_token_ids[:, :-1]
    x_lbe = jnp.take(emb, input_tok, axis=0).transpose(1, 0, 2)
    bih8 = jnp.broadcast_to(b_ih[None, :], (8, 3 * _H))
    bhh8 = jnp.broadcast_to(b_hh[None, :], (8, 3 * _H))
    os_lbh = _pc(
        _gru_body,
        grid=(_L,),
        in_specs=[
            pl.BlockSpec((1, _B, _E), lambda l: (l, 0, 0)),
            pl.BlockSpec((_B, _H), lambda l: (0, 0)),
            pl.BlockSpec((_E, 3 * _H), lambda l: (0, 0)),
            pl.BlockSpec((_H, 3 * _H), lambda l: (0, 0)),
            pl.BlockSpec((8, 3 * _H), lambda l: (0, 0)),
            pl.BlockSpec((8, 3 * _H), lambda l: (0, 0)),
        ],
        out_specs=pl.BlockSpec((1, _B, _H), lambda l: (l, 0, 0)),
        out_shape=jax.ShapeDtypeStruct((_L, _B, _H), f32),
        scratch_shapes=[pltpu.VMEM((_B, _H), f32)],
    )(x_lbe, initial_states, w_ih.T, w_hh.T, bih8, bhh8)

    org_lanes = jnp.broadcast_to(
        input_memories_origin_idx[:, None], (_NI, _B))
    osf = os_lbh.transpose(1, 0, 2).reshape(_B, _L * _H)
    s_std, s_cpy = _pc(
        _scores_body,
        grid=(_NI // _TI,),
        in_specs=[
            pl.BlockSpec((_TI, 2 * _H), lambda t: (t, 0)),
            pl.BlockSpec((_TI, _B), lambda t: (t, 0)),
            pl.BlockSpec((_B, _L * _H), lambda t: (0, 0)),
        ],
        out_specs=[
            pl.BlockSpec((_TI, _L), lambda t: (t, 0)),
            pl.BlockSpec((_TI, _L), lambda t: (t, 0)),
        ],
        out_shape=[
            jax.ShapeDtypeStruct((_NI, _L), f32),
            jax.ShapeDtypeStruct((_NI, _L), f32),
        ],
    )(reps, org_lanes, osf)

    so_lbh, tc_lb = _pc(
        _segstats_body,
        in_specs=[
            pl.BlockSpec((_NI, _L), lambda: (0, 0)),
            pl.BlockSpec((_NI, _L), lambda: (0, 0)),
            pl.BlockSpec((_NI, _B), lambda: (0, 0)),
            pl.BlockSpec((_NI, _H), lambda: (0, 0)),
        ],
        out_specs=[
            pl.BlockSpec((_L, _B, _H), lambda: (0, 0, 0)),
            pl.BlockSpec((_L, _B), lambda: (0, 0)),
        ],
        out_shape=[
            jax.ShapeDtypeStruct((_L, _B, _H), f32),
            jax.ShapeDtypeStruct((_L, _B), f32),
        ],
        scratch_shapes=[
            pltpu.VMEM((_L, _B), f32),
            pltpu.VMEM((_L, _B), f32),
        ],
    )(s_std, s_cpy, org_lanes, reps[:, :_H])

    cat = jnp.concatenate(
        [so_lbh.transpose(1, 0, 2), os_lbh.transpose(1, 0, 2)],
        axis=-1).reshape(_BL, 2 * _H)
    gold = target_token_ids[:, 1:].reshape(_BL)
    gold_lanes = jnp.broadcast_to(gold[:, None], (_BL, 128))
    tc_bl = tc_lb.T.reshape(_BL)
    tc_lanes = jnp.broadcast_to(tc_bl[:, None], (_BL, 128))
    emb_p = jnp.concatenate(
        [emb, jnp.zeros((_VP - _V, _E), f32)], axis=0)
    bias_p = jnp.concatenate(
        [vocab_bias, jnp.full((_VP - _V,), -1e30, f32)])
    bias8 = jnp.broadcast_to(bias_p[None, :], (8, _VP))
    norm_w, gold_w = _pc(
        _vocab_body,
        grid=(_VP // _VT,),
        in_specs=[
            pl.BlockSpec((_BL, 2 * _H), lambda v: (0, 0)),
            pl.BlockSpec((2 * _H, _E), lambda v: (0, 0)),
            pl.BlockSpec((_VT, _E), lambda v: (v, 0)),
            pl.BlockSpec((8, _VT), lambda v: (0, v)),
            pl.BlockSpec((_BL, 128), lambda v: (0, 0)),
            pl.BlockSpec((_BL, 128), lambda v: (0, 0)),
        ],
        out_specs=[
            pl.BlockSpec((_BL, 128), lambda v: (0, 0)),
            pl.BlockSpec((_BL, 128), lambda v: (0, 0)),
        ],
        out_shape=[
            jax.ShapeDtypeStruct((_BL, 128), f32),
            jax.ShapeDtypeStruct((_BL, 128), f32),
        ],
        scratch_shapes=[
            pltpu.VMEM((_BL, _H), f32),
            pltpu.VMEM((_BL, 128), f32),
            pltpu.VMEM((_BL, 128), f32),
            pltpu.VMEM((_BL, 128), f32),
        ],
    )(cat, hidden_to_vocab, emb_p, bias8, gold_lanes, tc_lanes)
    norm_bl = norm_w[:, 0].reshape(_B, _L)

    clp = _pc(
        _clp_body,
        in_specs=[
            pl.BlockSpec((_NI, _L), lambda: (0, 0)),
            pl.BlockSpec((_NI, _B), lambda: (0, 0)),
            pl.BlockSpec((_B, _L), lambda: (0, 0)),
        ],
        out_specs=pl.BlockSpec((_NI, 128), lambda: (0, 0)),
        out_shape=jax.ShapeDtypeStruct((_NI, 128), f32),
    )(s_cpy, org_lanes, norm_bl)

    selr = _sc_gather(clp, copyable_elements_idxs // _L)
    col_lanes = jnp.broadcast_to(
        (copyable_elements_idxs % _L)[:, None], (_NC, 128))
    sidx_lanes = jnp.broadcast_to(
        copyable_elements_sample_idxs[:, None], (_NC, 128))

    cc_w, nv_w = _pc(
        _copyseg_body,
        grid=(2, _NC // _CT),
        in_specs=[
            pl.BlockSpec((_CT, 128), lambda p, t: (t, 0)),
            pl.BlockSpec((_CT, 128), lambda p, t: (t, 0)),
            pl.BlockSpec((_CT, 128), lambda p, t: (t, 0)),
        ],
        out_specs=[
            pl.BlockSpec((8, _BL), lambda p, t: (0, 0)),
            pl.BlockSpec((8, _BL), lambda p, t: (0, 0)),
        ],
        out_shape=[
            jax.ShapeDtypeStruct((8, _BL), f32),
            jax.ShapeDtypeStruct((8, _BL), f32),
        ],
        scratch_shapes=[
            pltpu.VMEM((8, _BL), f32),
            pltpu.VMEM((8, _BL), f32),
            pltpu.VMEM((8, _BL), f32),
        ],
    )(selr, col_lanes, sidx_lanes)

    glp_bl = (gold_w[:, 0] - norm_w[:, 0]).reshape(_B, _L)
    cc_bl = cc_w[0].reshape(_B, _L)
    nv_bl = nv_w[0].reshape(_B, _L)
    tt = target_token_ids[:, 1:]
    len_b = jnp.broadcast_to(target_lengths[:, None], (_B, _L))
    out = _pc(
        _loss_body,
        in_specs=[
            pl.BlockSpec((_B, _L), lambda: (0, 0)),
            pl.BlockSpec((_B, _L), lambda: (0, 0)),
            pl.BlockSpec((_B, _L), lambda: (0, 0)),
            pl.BlockSpec((_B, _L), lambda: (0, 0)),
            pl.BlockSpec((_B, _L), lambda: (0, 0)),
        ],
        out_specs=pl.BlockSpec((8, 128), lambda: (0, 0)),
        out_shape=jax.ShapeDtypeStruct((8, 128), f32),
    )(glp_bl, cc_bl, nv_bl, tt, len_b)
    return out[0, 0]
